# Initial kernel scaffold; baseline (speedup 1.0000x reference)
import jax
import jax.numpy as jnp
from jax.experimental import pallas as pl


def kernel(x, edge_index_0, edge_index_1, edge_index_2, pos_edge_index, neg_edge_index, W_self_0, W_neigh_0, b_0, W_self_1, W_neigh_1, b_1, W_self_2, W_neigh_2, b_2, P_W1, P_b1, P_W2, P_b2, P_W3, P_b3):
    raise NotImplementedError("write your pallas kernel here")


# speedup vs baseline: 3.8840x; 3.8840x over previous
import functools

import jax
import jax.numpy as jnp
from jax import lax
from jax.experimental import pallas as pl
from jax.experimental.pallas import tpu as pltpu
from jax.experimental.pallas import tpu_sc as plsc

N = 10000
D = 128
E = 320000

NC = 2
NS = 16
NW = NC * NS

EPW = E // NW
CE = 80
NCHUNK = EPW // CE
NV = CE // 16

RCH = 80
NRCH = N // RCH
RCH_PER_TILE = -(-NRCH // NS)

NP2 = 10240

QH = 40960
QPW = 81920 // NW
QC = 128
QNCHUNK = QPW // QC


def _seg_body(h_hbm, src_hbm, dst_hbm, zacc_hbm, zdeg_hbm,
              agg_out, deg_out,
              sidx, didx, rows, degacc, acc_sp, sem):
    cid = lax.axis_index("c")
    sid = lax.axis_index("s")
    wid = cid * NS + sid

    pltpu.sync_copy(zacc_hbm, rows)
    pltpu.sync_copy(zdeg_hbm, degacc)

    def zinit(k, carry):
        j = sid + k * NS

        @pl.when(j < NRCH)
        def _():
            pltpu.sync_copy(rows, acc_sp.at[pl.ds(j * RCH, RCH)])

        return carry

    lax.fori_loop(0, RCH_PER_TILE, zinit, 0)
    plsc.subcore_barrier()

    ebase = wid * EPW
    vone = jnp.ones((16,), jnp.float32)

    def chunk(j, carry):
        b = ebase + j * CE
        pltpu.sync_copy(src_hbm.at[pl.ds(b, CE)], sidx)
        pltpu.sync_copy(dst_hbm.at[pl.ds(b, CE)], didx)
        pltpu.async_copy(h_hbm.at[sidx], rows, sem).wait()
        pltpu.sync_copy(rows, acc_sp.at[didx], add=True)
        for k in range(NV):
            idx = didx[pl.ds(k * 16, 16)]
            plsc.addupdate_scatter(degacc, [idx], vone)
        return carry

    lax.fori_loop(0, NCHUNK, chunk, 0)
    plsc.subcore_barrier()

    def flush(k, carry):
        j = sid + k * NS

        @pl.when(j < NRCH)
        def _():
            o = j * RCH
            pltpu.sync_copy(acc_sp.at[pl.ds(o, RCH)], rows)
            pltpu.sync_copy(rows, agg_out.at[pl.ds(cid * N + o, RCH)])

        return carry

    lax.fori_loop(0, RCH_PER_TILE, flush, 0)
    pltpu.sync_copy(degacc, deg_out.at[wid])


@jax.jit
def _seg_sum(h, src, dst, zacc, zdeg):
    mesh = plsc.VectorSubcoreMesh(core_axis_name="c", subcore_axis_name="s")
    return pl.kernel(
        _seg_body,
        out_type=[
            jax.ShapeDtypeStruct((NC * N, D), jnp.float32),
            jax.ShapeDtypeStruct((NW, NP2), jnp.float32),
        ],
        mesh=mesh,
        compiler_params=pltpu.CompilerParams(needs_layout_passes=False),
        scratch_types=[
            pltpu.VMEM((CE,), jnp.int32),
            pltpu.VMEM((CE,), jnp.int32),
            pltpu.VMEM((CE, D), jnp.float32),
            pltpu.VMEM((NP2,), jnp.float32),
            pltpu.VMEM_SHARED((N, D), jnp.float32),
            pltpu.SemaphoreType.DMA,
        ],
    )(h, src, dst, zacc, zdeg)


def _gather_body(h_hbm, idx_hbm, u_out, v_out, qidx, rows, sem):
    cid = lax.axis_index("c")
    sid = lax.axis_index("s")
    wid = cid * NS + sid

    @pl.when(wid < NW // 2)
    def _():
        base = wid * QPW

        def chunk(j, carry):
            b = base + j * QC
            pltpu.sync_copy(idx_hbm.at[pl.ds(b, QC)], qidx)
            pltpu.async_copy(h_hbm.at[qidx], rows, sem).wait()
            pltpu.sync_copy(rows, u_out.at[pl.ds(b, QC)])
            return carry

        lax.fori_loop(0, QNCHUNK, chunk, 0)

    @pl.when(wid >= NW // 2)
    def _():
        base = (wid - NW // 2) * QPW

        def chunk(j, carry):
            b = base + j * QC
            pltpu.sync_copy(idx_hbm.at[pl.ds(QH + b, QC)], qidx)
            pltpu.async_copy(h_hbm.at[qidx], rows, sem).wait()
            pltpu.sync_copy(rows, v_out.at[pl.ds(b, QC)])
            return carry

        lax.fori_loop(0, QNCHUNK, chunk, 0)


@jax.jit
def _pair_gather(h, qidx):
    mesh = plsc.VectorSubcoreMesh(core_axis_name="c", subcore_axis_name="s")
    return pl.kernel(
        _gather_body,
        out_type=[
            jax.ShapeDtypeStruct((QH, D), jnp.float32),
            jax.ShapeDtypeStruct((QH, D), jnp.float32),
        ],
        mesh=mesh,
        scratch_types=[
            pltpu.VMEM((QC,), jnp.int32),
            pltpu.VMEM((QC, D), jnp.float32),
            pltpu.SemaphoreType.DMA,
        ],
    )(h, qidx)


def _degred_body(dp_ref, o_ref):
    o_ref[...] = jnp.sum(dp_ref[...], axis=0).reshape(-1, 1)


DGB = 1024


@jax.jit
def _deg_reduce(deg_parts):
    grid = NP2 // DGB
    return pl.pallas_call(
        _degred_body,
        grid=(grid,),
        in_specs=[pl.BlockSpec((NW, DGB), lambda i: (0, i))],
        out_specs=pl.BlockSpec((DGB, 1), lambda i: (i, 0)),
        out_shape=jax.ShapeDtypeStruct((NP2, 1), jnp.float32),
    )(deg_parts)


def _dense_body(parts_ref, deg_ref, h_ref, ws_ref, wn_ref, b_ref, o_ref, *, relu):
    agg = parts_ref[0] + parts_ref[1]
    mean = agg / jnp.maximum(deg_ref[...], 1.0)
    out = (jnp.dot(h_ref[...], ws_ref[...], preferred_element_type=jnp.float32)
           + jnp.dot(mean, wn_ref[...], preferred_element_type=jnp.float32)
           + b_ref[...])
    o_ref[...] = jnp.maximum(out, 0.0) if relu else out


DR = 1000


@functools.partial(jax.jit, static_argnames=("relu",))
def _dense(parts, deg, h, ws, wn, b, relu):
    grid = N // DR
    return pl.pallas_call(
        functools.partial(_dense_body, relu=relu),
        grid=(grid,),
        in_specs=[
            pl.BlockSpec((NC, DR, D), lambda i: (0, i, 0)),
            pl.BlockSpec((DR, 1), lambda i: (i, 0)),
            pl.BlockSpec((DR, D), lambda i: (i, 0)),
            pl.BlockSpec((D, D), lambda i: (0, 0)),
            pl.BlockSpec((D, D), lambda i: (0, 0)),
            pl.BlockSpec((1, D), lambda i: (0, 0)),
        ],
        out_specs=pl.BlockSpec((DR, D), lambda i: (i, 0)),
        out_shape=jax.ShapeDtypeStruct((N, D), jnp.float32),
    )(parts, deg, h, ws, wn, b)


def _pred_body(u_ref, v_ref, w1, b1, w2, b2, w3, b3, o_ref):
    z = u_ref[...] * v_ref[...]
    a = jnp.maximum(jnp.dot(z, w1[...], preferred_element_type=jnp.float32) + b1[...], 0.0)
    a = jnp.maximum(jnp.dot(a, w2[...], preferred_element_type=jnp.float32) + b2[...], 0.0)
    o_ref[...] = jnp.dot(a, w3[...], preferred_element_type=jnp.float32) + b3[...]


PR = 512


@jax.jit
def _pred(u, v, w1, b1, w2, b2, w3, b3):
    grid = QH // PR
    return pl.pallas_call(
        _pred_body,
        grid=(grid,),
        in_specs=[
            pl.BlockSpec((PR, D), lambda i: (i, 0)),
            pl.BlockSpec((PR, D), lambda i: (i, 0)),
            pl.BlockSpec((D, D), lambda i: (0, 0)),
            pl.BlockSpec((1, D), lambda i: (0, 0)),
            pl.BlockSpec((D, D), lambda i: (0, 0)),
            pl.BlockSpec((1, D), lambda i: (0, 0)),
            pl.BlockSpec((D, 1), lambda i: (0, 0)),
            pl.BlockSpec((1, 1), lambda i: (0, 0)),
        ],
        out_specs=pl.BlockSpec((PR, 1), lambda i: (i, 0)),
        out_shape=jax.ShapeDtypeStruct((QH, 1), jnp.float32),
    )(u, v, w1, b1, w2, b2, w3, b3)


def kernel(x, edge_index_0, edge_index_1, edge_index_2, pos_edge_index, neg_edge_index,
           W_self_0, W_neigh_0, b_0, W_self_1, W_neigh_1, b_1, W_self_2, W_neigh_2, b_2,
           P_W1, P_b1, P_W2, P_b2, P_W3, P_b3):
    zacc = jnp.zeros((RCH, D), jnp.float32)
    zdeg = jnp.zeros((NP2,), jnp.float32)

    h = x
    layers = [
        (edge_index_0, W_self_0, W_neigh_0, b_0),
        (edge_index_1, W_self_1, W_neigh_1, b_1),
        (edge_index_2, W_self_2, W_neigh_2, b_2),
    ]
    for i, (ei, ws, wn, b) in enumerate(layers):
        agg_f, deg_parts = _seg_sum(h, ei[0], ei[1], zacc, zdeg)
        parts = agg_f.reshape(NC, N, D)
        deg = _deg_reduce(deg_parts)[:N]
        h = _dense(parts, deg, h, ws, wn, b.reshape(1, D), relu=(i < 2))

    pad = jnp.zeros((QH - 40000,), jnp.int32)
    qidx = jnp.concatenate([
        pos_edge_index[0], neg_edge_index[0], pad,
        pos_edge_index[1], neg_edge_index[1], pad,
    ])
    u, v = _pair_gather(h, qidx)
    out = _pred(u, v, P_W1, P_b1.reshape(1, D), P_W2, P_b2.reshape(1, D),
                P_W3, P_b3.reshape(1, 1))
    return out[:20000], out[20000:40000]


# speedup vs baseline: 5.1240x; 1.3193x over previous
import functools

import jax
import jax.numpy as jnp
from jax import lax
from jax.experimental import pallas as pl
from jax.experimental.pallas import tpu as pltpu
from jax.experimental.pallas import tpu_sc as plsc

N = 10000
D = 128
E = 320000

NC = 2
NS = 16
NW = NC * NS

EPW = E // NW
CE = 80
NV = CE // 16
NBUF = 3
EPW_P = 10080
NCHUNK = EPW_P // CE
NGROUP = NCHUNK // NBUF
NACC = N + 16

RCH = 80
NRCH = N // RCH
RCH_PER_TILE = -(-NRCH // NS)

NP2 = 10240

QH = 40960
QPW = 81920 // NW
QC = 128
QNCHUNK = QPW // QC


def _seg_body(h_hbm, src_hbm, dst_hbm, zacc_hbm, zdeg_hbm,
              agg_out, deg_out,
              sblk, dblk, degacc, acc_sp,
              r0, r1, r2, g0, g1, g2, s0, s1, s2, i0, i1):
    rows = (r0, r1, r2)
    gsem = (g0, g1, g2)
    ssem = (s0, s1, s2)
    isem = (i0, i1)
    cid = lax.axis_index("c")
    sid = lax.axis_index("s")
    wid = cid * NS + sid

    gbase = wid * NGROUP
    pltpu.sync_copy(src_hbm.at[gbase], sblk.at[0])
    pltpu.sync_copy(dst_hbm.at[gbase], dblk.at[0])
    pltpu.async_copy(src_hbm.at[gbase + 1], sblk.at[1], isem[1])
    pltpu.async_copy(dst_hbm.at[gbase + 1], dblk.at[1], isem[1])
    pltpu.sync_copy(zdeg_hbm, degacc)

    pltpu.sync_copy(zacc_hbm, r0)

    def zinit(k, carry):
        j = sid + k * NS

        @pl.when(j < NRCH)
        def _():
            pltpu.sync_copy(r0, acc_sp.at[pl.ds(j * RCH, RCH)])

        return carry

    lax.fori_loop(0, RCH_PER_TILE, zinit, 0)
    plsc.subcore_barrier()

    vone = jnp.ones((16,), jnp.float32)

    for b in range(NBUF):
        pltpu.async_copy(h_hbm.at[sblk.at[0, b]], rows[b], gsem[b])

    def run_group(g, slot):
        nxt = 1 - slot
        for b in range(NBUF):
            pltpu.make_async_copy(h_hbm.at[sblk.at[slot, b]], rows[b],
                                  gsem[b]).wait()
            pltpu.async_copy(rows[b], acc_sp.at[dblk.at[slot, b]], ssem[b],
                             add=True)
            for k in range(NV):
                idx = dblk[slot, b, pl.ds(k * 16, 16)]
                plsc.addupdate_scatter(degacc, [idx], vone)

        @pl.when(g < NGROUP - 1)
        def _():
            pltpu.make_async_copy(src_hbm.at[gbase], sblk.at[nxt],
                                  isem[nxt]).wait()
            pltpu.make_async_copy(dst_hbm.at[gbase], dblk.at[nxt],
                                  isem[nxt]).wait()

        for b in range(NBUF):
            pltpu.make_async_copy(rows[b], acc_sp.at[dblk.at[slot, b]],
                                  ssem[b]).wait()

            @pl.when(g < NGROUP - 1)
            def _():
                pltpu.async_copy(h_hbm.at[sblk.at[nxt, b]], rows[b], gsem[b])

        @pl.when(g < NGROUP - 2)
        def _():
            pltpu.async_copy(src_hbm.at[gbase + g + 2], sblk.at[slot],
                             isem[slot])
            pltpu.async_copy(dst_hbm.at[gbase + g + 2], dblk.at[slot],
                             isem[slot])

    def pair(t, carry):
        run_group(2 * t, 0)
        run_group(2 * t + 1, 1)
        return carry

    lax.fori_loop(0, NGROUP // 2, pair, 0)
    plsc.subcore_barrier()

    for k in range(RCH_PER_TILE):
        j = sid + k * NS
        b = k % NBUF
        if k >= NBUF:
            pltpu.make_async_copy(rows[b], agg_out.at[pl.ds(0, RCH)], gsem[b]).wait()

        @pl.when(j < NRCH)
        def _():
            o = j * RCH
            pltpu.sync_copy(acc_sp.at[pl.ds(o, RCH)], rows[b])
            pltpu.async_copy(rows[b], agg_out.at[pl.ds(cid * N + o, RCH)], gsem[b])

    for k in range(max(0, RCH_PER_TILE - NBUF), RCH_PER_TILE):
        j = sid + k * NS
        b = k % NBUF

        @pl.when(j < NRCH)
        def _():
            pltpu.make_async_copy(rows[b], agg_out.at[pl.ds(0, RCH)], gsem[b]).wait()

    pltpu.sync_copy(degacc, deg_out.at[wid])


@jax.jit
def _seg_sum(h, src, dst, zacc, zdeg):
    mesh = plsc.VectorSubcoreMesh(core_axis_name="c", subcore_axis_name="s")
    return pl.kernel(
        _seg_body,
        out_type=[
            jax.ShapeDtypeStruct((NC * N, D), jnp.float32),
            jax.ShapeDtypeStruct((NW, NP2), jnp.float32),
        ],
        mesh=mesh,
        compiler_params=pltpu.CompilerParams(needs_layout_passes=False),
        scratch_types=[
            pltpu.VMEM((2, 8, CE), jnp.int32),
            pltpu.VMEM((2, 8, CE), jnp.int32),
            pltpu.VMEM((NP2,), jnp.float32),
            pltpu.VMEM_SHARED((NACC, D), jnp.float32),
        ] + [pltpu.VMEM((CE, D), jnp.float32)] * NBUF
          + [pltpu.SemaphoreType.DMA] * (2 * NBUF + 2),
    )(h, src, dst, zacc, zdeg)


NBUFQ = 4
QNGROUP = QNCHUNK // NBUFQ


def _gather_body(h_hbm, idx_hbm, u_out, v_out, qidx,
                 q0, q1, q2, q3, gg0, gg1, gg2, gg3, ww0, ww1, ww2, ww3):
    bufs = (q0, q1, q2, q3)
    gsem = (gg0, gg1, gg2, gg3)
    wsem = (ww0, ww1, ww2, ww3)
    cid = lax.axis_index("c")
    sid = lax.axis_index("s")
    wid = cid * NS + sid

    pltpu.sync_copy(idx_hbm.at[wid], qidx)

    def run(out_ref, base):
        for b in range(NBUFQ):
            pltpu.async_copy(h_hbm.at[qidx.at[b]], bufs[b], gsem[b])

        def group(g, carry):
            gb = g * NBUFQ
            for b in range(NBUFQ):
                j = gb + b
                pltpu.make_async_copy(h_hbm.at[qidx.at[j]], bufs[b], gsem[b]).wait()
                pltpu.async_copy(bufs[b], out_ref.at[pl.ds(base + j * QC, QC)],
                                 wsem[b])
            for b in range(NBUFQ):
                j = gb + b
                pltpu.make_async_copy(bufs[b], out_ref.at[pl.ds(0, QC)],
                                      wsem[b]).wait()

                @pl.when(g < QNGROUP - 1)
                def _():
                    pltpu.async_copy(h_hbm.at[qidx.at[j + NBUFQ]], bufs[b], gsem[b])

            return carry

        lax.fori_loop(0, QNGROUP, group, 0)

    @pl.when(wid < NW // 2)
    def _():
        run(u_out, wid * QPW)

    @pl.when(wid >= NW // 2)
    def _():
        run(v_out, (wid - NW // 2) * QPW)


@jax.jit
def _pair_gather(h, qidx):
    mesh = plsc.VectorSubcoreMesh(core_axis_name="c", subcore_axis_name="s")
    return pl.kernel(
        _gather_body,
        out_type=[
            jax.ShapeDtypeStruct((QH, D), jnp.float32),
            jax.ShapeDtypeStruct((QH, D), jnp.float32),
        ],
        mesh=mesh,
        scratch_types=[
            pltpu.VMEM((QNCHUNK, QC), jnp.int32),
        ] + [pltpu.VMEM((QC, D), jnp.float32)] * NBUFQ
          + [pltpu.SemaphoreType.DMA] * (2 * NBUFQ),
    )(h, qidx)


def _degred_body(dp_ref, o_ref):
    o_ref[...] = jnp.sum(dp_ref[...], axis=0).reshape(-1, 1)


DGB = 1024


@jax.jit
def _deg_reduce(deg_parts):
    grid = NP2 // DGB
    return pl.pallas_call(
        _degred_body,
        grid=(grid,),
        in_specs=[pl.BlockSpec((NW, DGB), lambda i: (0, i))],
        out_specs=pl.BlockSpec((DGB, 1), lambda i: (i, 0)),
        out_shape=jax.ShapeDtypeStruct((NP2, 1), jnp.float32),
    )(deg_parts)


def _dense_body(parts_ref, deg_ref, h_ref, ws_ref, wn_ref, b_ref, o_ref, *, relu):
    agg = parts_ref[0] + parts_ref[1]
    mean = agg / jnp.maximum(deg_ref[...], 1.0)
    out = (jnp.dot(h_ref[...], ws_ref[...], preferred_element_type=jnp.float32)
           + jnp.dot(mean, wn_ref[...], preferred_element_type=jnp.float32)
           + b_ref[...])
    o_ref[...] = jnp.maximum(out, 0.0) if relu else out


DR = 1000


@functools.partial(jax.jit, static_argnames=("relu",))
def _dense(parts, deg, h, ws, wn, b, relu):
    grid = N // DR
    return pl.pallas_call(
        functools.partial(_dense_body, relu=relu),
        grid=(grid,),
        in_specs=[
            pl.BlockSpec((NC, DR, D), lambda i: (0, i, 0)),
            pl.BlockSpec((DR, 1), lambda i: (i, 0)),
            pl.BlockSpec((DR, D), lambda i: (i, 0)),
            pl.BlockSpec((D, D), lambda i: (0, 0)),
            pl.BlockSpec((D, D), lambda i: (0, 0)),
            pl.BlockSpec((1, D), lambda i: (0, 0)),
        ],
        out_specs=pl.BlockSpec((DR, D), lambda i: (i, 0)),
        out_shape=jax.ShapeDtypeStruct((N, D), jnp.float32),
    )(parts, deg, h, ws, wn, b)


def _pred_body(u_ref, v_ref, w1, b1, w2, b2, w3, b3, o_ref):
    z = u_ref[...] * v_ref[...]
    a = jnp.maximum(jnp.dot(z, w1[...], preferred_element_type=jnp.float32) + b1[...], 0.0)
    a = jnp.maximum(jnp.dot(a, w2[...], preferred_element_type=jnp.float32) + b2[...], 0.0)
    o_ref[...] = jnp.dot(a, w3[...], preferred_element_type=jnp.float32) + b3[...]


PR = 512


@jax.jit
def _pred(u, v, w1, b1, w2, b2, w3, b3):
    grid = QH // PR
    return pl.pallas_call(
        _pred_body,
        grid=(grid,),
        in_specs=[
            pl.BlockSpec((PR, D), lambda i: (i, 0)),
            pl.BlockSpec((PR, D), lambda i: (i, 0)),
            pl.BlockSpec((D, D), lambda i: (0, 0)),
            pl.BlockSpec((1, D), lambda i: (0, 0)),
            pl.BlockSpec((D, D), lambda i: (0, 0)),
            pl.BlockSpec((1, D), lambda i: (0, 0)),
            pl.BlockSpec((D, 1), lambda i: (0, 0)),
            pl.BlockSpec((1, 1), lambda i: (0, 0)),
        ],
        out_specs=pl.BlockSpec((PR, 1), lambda i: (i, 0)),
        out_shape=jax.ShapeDtypeStruct((QH, 1), jnp.float32),
    )(u, v, w1, b1, w2, b2, w3, b3)


def kernel(x, edge_index_0, edge_index_1, edge_index_2, pos_edge_index, neg_edge_index,
           W_self_0, W_neigh_0, b_0, W_self_1, W_neigh_1, b_1, W_self_2, W_neigh_2, b_2,
           P_W1, P_b1, P_W2, P_b2, P_W3, P_b3):
    zacc = jnp.zeros((RCH, D), jnp.float32)
    zdeg = jnp.zeros((NP2,), jnp.float32)

    h = x
    layers = [
        (edge_index_0, W_self_0, W_neigh_0, b_0),
        (edge_index_1, W_self_1, W_neigh_1, b_1),
        (edge_index_2, W_self_2, W_neigh_2, b_2),
    ]
    epad = EPW_P - EPW
    rpad = ((0, 0), (0, 0), (0, 8 - NBUF), (0, 0))
    for i, (ei, ws, wn, b) in enumerate(layers):
        src3 = jnp.pad(
            jnp.pad(ei[0].reshape(NW, EPW), ((0, 0), (0, epad))
                    ).reshape(NW, NGROUP, NBUF, CE), rpad
        ).reshape(NW * NGROUP, 8, CE)
        dst3 = jnp.pad(
            jnp.pad(ei[1].reshape(NW, EPW), ((0, 0), (0, epad)),
                    constant_values=N).reshape(NW, NGROUP, NBUF, CE), rpad,
            constant_values=N,
        ).reshape(NW * NGROUP, 8, CE)
        agg_f, deg_parts = _seg_sum(h, src3, dst3, zacc, zdeg)
        parts = agg_f.reshape(NC, N, D)
        deg = _deg_reduce(deg_parts)[:N]
        h = _dense(parts, deg, h, ws, wn, b.reshape(1, D), relu=(i < 2))

    pad = jnp.zeros((QH - 40000,), jnp.int32)
    qidx = jnp.concatenate([
        pos_edge_index[0], neg_edge_index[0], pad,
        pos_edge_index[1], neg_edge_index[1], pad,
    ]).reshape(NW, QNCHUNK, QC)
    u, v = _pair_gather(h, qidx)
    out = _pred(u, v, P_W1, P_b1.reshape(1, D), P_W2, P_b2.reshape(1, D),
                P_W3, P_b3.reshape(1, 1))
    return out[:20000], out[20000:40000]


# speedup vs baseline: 5.2280x; 1.0203x over previous
import functools

import jax
import jax.numpy as jnp
from jax import lax
from jax.experimental import pallas as pl
from jax.experimental.pallas import tpu as pltpu
from jax.experimental.pallas import tpu_sc as plsc

N = 10000
D = 128
E = 320000

NC = 2
NS = 16
NW = NC * NS

EPW = E // NW
CE = 80
NV = CE // 16
NBUF = 3
EPW_P = 10080
NCHUNK = EPW_P // CE
NGROUP = NCHUNK // NBUF
NACC = N + 16

RCH = 80
NRCH = N // RCH
RCH_PER_TILE = -(-NRCH // NS)

NP2 = 10240

QH = 40960
QPW = 81920 // NW
QC = 128
QNCHUNK = QPW // QC


def _seg_body(h_hbm, src_hbm, dst_hbm, zacc_hbm, zdeg_hbm,
              agg_out, deg_out,
              sblk, dblk, degacc, acc_sp,
              r0, r1, r2, g0, g1, g2, s0, s1, s2, i0, i1):
    rows = (r0, r1, r2)
    gsem = (g0, g1, g2)
    ssem = (s0, s1, s2)
    isem = (i0, i1)
    cid = lax.axis_index("c")
    sid = lax.axis_index("s")
    wid = cid * NS + sid

    gbase = wid * NGROUP
    pltpu.sync_copy(src_hbm.at[gbase], sblk.at[0])
    pltpu.sync_copy(dst_hbm.at[gbase], dblk.at[0])
    pltpu.async_copy(src_hbm.at[gbase + 1], sblk.at[1], isem[1])
    pltpu.async_copy(dst_hbm.at[gbase + 1], dblk.at[1], isem[1])
    pltpu.sync_copy(zdeg_hbm, degacc)

    pltpu.sync_copy(zacc_hbm, r0)

    def zinit(k, carry):
        j = sid + k * NS

        @pl.when(j < NRCH)
        def _():
            pltpu.sync_copy(r0, acc_sp.at[pl.ds(j * RCH, RCH)])

        return carry

    lax.fori_loop(0, RCH_PER_TILE, zinit, 0)
    plsc.subcore_barrier()

    vone = jnp.ones((16,), jnp.float32)

    for b in range(NBUF):
        pltpu.async_copy(h_hbm.at[sblk.at[0, b]], rows[b], gsem[b])

    def run_group(g, slot):
        nxt = 1 - slot
        for b in range(NBUF):
            pltpu.make_async_copy(h_hbm.at[sblk.at[slot, b]], rows[b],
                                  gsem[b]).wait()
            pltpu.async_copy(rows[b], acc_sp.at[dblk.at[slot, b]], ssem[b],
                             add=True)
            for k in range(NV):
                idx = dblk[slot, b, pl.ds(k * 16, 16)]
                plsc.addupdate_scatter(degacc, [idx], vone)

        @pl.when(g < NGROUP - 1)
        def _():
            pltpu.make_async_copy(src_hbm.at[gbase], sblk.at[nxt],
                                  isem[nxt]).wait()
            pltpu.make_async_copy(dst_hbm.at[gbase], dblk.at[nxt],
                                  isem[nxt]).wait()

        for b in range(NBUF):
            pltpu.make_async_copy(rows[b], acc_sp.at[dblk.at[slot, b]],
                                  ssem[b]).wait()

            @pl.when(g < NGROUP - 1)
            def _():
                pltpu.async_copy(h_hbm.at[sblk.at[nxt, b]], rows[b], gsem[b])

        @pl.when(g < NGROUP - 2)
        def _():
            pltpu.async_copy(src_hbm.at[gbase + g + 2], sblk.at[slot],
                             isem[slot])
            pltpu.async_copy(dst_hbm.at[gbase + g + 2], dblk.at[slot],
                             isem[slot])

    def pair(t, carry):
        run_group(2 * t, 0)
        run_group(2 * t + 1, 1)
        return carry

    lax.fori_loop(0, NGROUP // 2, pair, 0)
    plsc.subcore_barrier()

    for k in range(RCH_PER_TILE):
        j = sid + k * NS
        b = k % NBUF
        if k >= NBUF:
            pltpu.make_async_copy(rows[b], agg_out.at[pl.ds(0, RCH)], gsem[b]).wait()

        @pl.when(j < NRCH)
        def _():
            o = j * RCH
            pltpu.sync_copy(acc_sp.at[pl.ds(o, RCH)], rows[b])
            pltpu.async_copy(rows[b], agg_out.at[pl.ds(cid * N + o, RCH)], gsem[b])

    for k in range(max(0, RCH_PER_TILE - NBUF), RCH_PER_TILE):
        j = sid + k * NS
        b = k % NBUF

        @pl.when(j < NRCH)
        def _():
            pltpu.make_async_copy(rows[b], agg_out.at[pl.ds(0, RCH)], gsem[b]).wait()

    pltpu.sync_copy(degacc, deg_out.at[wid])


@jax.jit
def _seg_sum(h, src, dst, zacc, zdeg):
    mesh = plsc.VectorSubcoreMesh(core_axis_name="c", subcore_axis_name="s")
    return pl.kernel(
        _seg_body,
        out_type=[
            jax.ShapeDtypeStruct((NC * N, D), jnp.float32),
            jax.ShapeDtypeStruct((NW, NP2), jnp.float32),
        ],
        mesh=mesh,
        compiler_params=pltpu.CompilerParams(needs_layout_passes=False),
        scratch_types=[
            pltpu.VMEM((2, 8, CE), jnp.int32),
            pltpu.VMEM((2, 8, CE), jnp.int32),
            pltpu.VMEM((NP2,), jnp.float32),
            pltpu.VMEM_SHARED((NACC, D), jnp.float32),
        ] + [pltpu.VMEM((CE, D), jnp.float32)] * NBUF
          + [pltpu.SemaphoreType.DMA] * (2 * NBUF + 2),
    )(h, src, dst, zacc, zdeg)


NBUFQ = 4
QNGROUP = QNCHUNK // NBUFQ


def _gather_body(h_hbm, idx_hbm, u_out, v_out, qidx,
                 q0, q1, q2, q3, gg0, gg1, gg2, gg3, ww0, ww1, ww2, ww3):
    bufs = (q0, q1, q2, q3)
    gsem = (gg0, gg1, gg2, gg3)
    wsem = (ww0, ww1, ww2, ww3)
    cid = lax.axis_index("c")
    sid = lax.axis_index("s")
    wid = cid * NS + sid

    pltpu.sync_copy(idx_hbm.at[wid], qidx)

    def run(out_ref, base):
        for b in range(NBUFQ):
            pltpu.async_copy(h_hbm.at[qidx.at[b]], bufs[b], gsem[b])

        def group(g, carry):
            gb = g * NBUFQ
            for b in range(NBUFQ):
                j = gb + b
                pltpu.make_async_copy(h_hbm.at[qidx.at[j]], bufs[b], gsem[b]).wait()
                pltpu.async_copy(bufs[b], out_ref.at[pl.ds(base + j * QC, QC)],
                                 wsem[b])
            for b in range(NBUFQ):
                j = gb + b
                pltpu.make_async_copy(bufs[b], out_ref.at[pl.ds(0, QC)],
                                      wsem[b]).wait()

                @pl.when(g < QNGROUP - 1)
                def _():
                    pltpu.async_copy(h_hbm.at[qidx.at[j + NBUFQ]], bufs[b], gsem[b])

            return carry

        lax.fori_loop(0, QNGROUP, group, 0)

    @pl.when(wid < NW // 2)
    def _():
        run(u_out, wid * QPW)

    @pl.when(wid >= NW // 2)
    def _():
        run(v_out, (wid - NW // 2) * QPW)


@jax.jit
def _pair_gather(h, qidx):
    mesh = plsc.VectorSubcoreMesh(core_axis_name="c", subcore_axis_name="s")
    return pl.kernel(
        _gather_body,
        out_type=[
            jax.ShapeDtypeStruct((QH, D), jnp.float32),
            jax.ShapeDtypeStruct((QH, D), jnp.float32),
        ],
        mesh=mesh,
        scratch_types=[
            pltpu.VMEM((QNCHUNK, QC), jnp.int32),
        ] + [pltpu.VMEM((QC, D), jnp.float32)] * NBUFQ
          + [pltpu.SemaphoreType.DMA] * (2 * NBUFQ),
    )(h, qidx)


def _dense_body(parts_ref, degp_ref, h_ref, ws_ref, wn_ref, b_ref, o_ref, *, relu):
    agg = parts_ref[0] + parts_ref[1]
    deg = jnp.sum(degp_ref[...], axis=1).reshape(-1, 1)
    mean = agg / jnp.maximum(deg, 1.0)
    out = (jnp.dot(h_ref[...], ws_ref[...], preferred_element_type=jnp.float32)
           + jnp.dot(mean, wn_ref[...], preferred_element_type=jnp.float32)
           + b_ref[...])
    o_ref[...] = jnp.maximum(out, 0.0) if relu else out


DR = 1000


@functools.partial(jax.jit, static_argnames=("relu",))
def _dense(parts, deg_parts, h, ws, wn, b, relu):
    grid = N // DR
    return pl.pallas_call(
        functools.partial(_dense_body, relu=relu),
        grid=(grid,),
        in_specs=[
            pl.BlockSpec((NC, DR, D), lambda i: (0, i, 0)),
            pl.BlockSpec((DR, NW), lambda i: (i, 0)),
            pl.BlockSpec((DR, D), lambda i: (i, 0)),
            pl.BlockSpec((D, D), lambda i: (0, 0)),
            pl.BlockSpec((D, D), lambda i: (0, 0)),
            pl.BlockSpec((1, D), lambda i: (0, 0)),
        ],
        out_specs=pl.BlockSpec((DR, D), lambda i: (i, 0)),
        out_shape=jax.ShapeDtypeStruct((N, D), jnp.float32),
    )(parts, deg_parts, h, ws, wn, b)


def _pred_body(u_ref, v_ref, w1, b1, w2, b2, w3, b3, o_ref):
    z = u_ref[...] * v_ref[...]
    a = jnp.maximum(jnp.dot(z, w1[...], preferred_element_type=jnp.float32) + b1[...], 0.0)
    a = jnp.maximum(jnp.dot(a, w2[...], preferred_element_type=jnp.float32) + b2[...], 0.0)
    o_ref[...] = jnp.dot(a, w3[...], preferred_element_type=jnp.float32) + b3[...]


PR = 512


@jax.jit
def _pred(u, v, w1, b1, w2, b2, w3, b3):
    grid = QH // PR
    return pl.pallas_call(
        _pred_body,
        grid=(grid,),
        in_specs=[
            pl.BlockSpec((PR, D), lambda i: (i, 0)),
            pl.BlockSpec((PR, D), lambda i: (i, 0)),
            pl.BlockSpec((D, D), lambda i: (0, 0)),
            pl.BlockSpec((1, D), lambda i: (0, 0)),
            pl.BlockSpec((D, D), lambda i: (0, 0)),
            pl.BlockSpec((1, D), lambda i: (0, 0)),
            pl.BlockSpec((D, 1), lambda i: (0, 0)),
            pl.BlockSpec((1, 1), lambda i: (0, 0)),
        ],
        out_specs=pl.BlockSpec((PR, 1), lambda i: (i, 0)),
        out_shape=jax.ShapeDtypeStruct((QH, 1), jnp.float32),
    )(u, v, w1, b1, w2, b2, w3, b3)


def kernel(x, edge_index_0, edge_index_1, edge_index_2, pos_edge_index, neg_edge_index,
           W_self_0, W_neigh_0, b_0, W_self_1, W_neigh_1, b_1, W_self_2, W_neigh_2, b_2,
           P_W1, P_b1, P_W2, P_b2, P_W3, P_b3):
    zacc = jnp.zeros((RCH, D), jnp.float32)
    zdeg = jnp.zeros((NP2,), jnp.float32)

    h = x
    layers = [
        (edge_index_0, W_self_0, W_neigh_0, b_0),
        (edge_index_1, W_self_1, W_neigh_1, b_1),
        (edge_index_2, W_self_2, W_neigh_2, b_2),
    ]
    epad = EPW_P - EPW
    rpad = ((0, 0), (0, 0), (0, 8 - NBUF), (0, 0))
    for i, (ei, ws, wn, b) in enumerate(layers):
        src3 = jnp.pad(
            jnp.pad(ei[0].reshape(NW, EPW), ((0, 0), (0, epad))
                    ).reshape(NW, NGROUP, NBUF, CE), rpad
        ).reshape(NW * NGROUP, 8, CE)
        dst3 = jnp.pad(
            jnp.pad(ei[1].reshape(NW, EPW), ((0, 0), (0, epad)),
                    constant_values=N).reshape(NW, NGROUP, NBUF, CE), rpad,
            constant_values=N,
        ).reshape(NW * NGROUP, 8, CE)
        agg_f, deg_parts = _seg_sum(h, src3, dst3, zacc, zdeg)
        parts = agg_f.reshape(NC, N, D)
        h = _dense(parts, deg_parts.T, h, ws, wn, b.reshape(1, D), relu=(i < 2))

    pad = jnp.zeros((QH - 40000,), jnp.int32)
    qidx = jnp.concatenate([
        pos_edge_index[0], neg_edge_index[0], pad,
        pos_edge_index[1], neg_edge_index[1], pad,
    ]).reshape(NW, QNCHUNK, QC)
    u, v = _pair_gather(h, qidx)
    out = _pred(u, v, P_W1, P_b1.reshape(1, D), P_W2, P_b2.reshape(1, D),
                P_W3, P_b3.reshape(1, 1))
    return out[:20000], out[20000:40000]
